# baseline (speedup 1.0000x reference)
import functools

import jax
import jax.numpy as jnp
from jax import lax
from jax.experimental import pallas as pl
from jax.experimental.pallas import tpu as pltpu
from jax.experimental.pallas import tpu_sc as plsc

N = 16384
D = 64
FD = 128
VOCAB = 1000000
CHUNK = 128


@functools.lru_cache(maxsize=1)
def _make_gather():
    info = plsc.get_sparse_core_info()
    nc, ns = info.num_cores, info.num_subcores
    nw = nc * ns
    bpw = N // nw
    nch = bpw // CHUNK
    mesh = plsc.VectorSubcoreMesh(core_axis_name="c", subcore_axis_name="s")

    @functools.partial(
        pl.kernel,
        mesh=mesh,
        out_type=jax.ShapeDtypeStruct((N, FD), jnp.float32),
        compiler_params=pltpu.CompilerParams(use_tc_tiling_on_sc=True),
        scratch_types=[
            pltpu.VMEM((nch, CHUNK), jnp.int32),
            pltpu.VMEM((bpw, FD), jnp.float32),
            pltpu.SemaphoreType.DMA,
        ],
    )
    def gather_k(table_hbm, idx_hbm, out_hbm, idx_v, rows_v, sem):
        wid = lax.axis_index("s") * nc + lax.axis_index("c")
        pltpu.sync_copy(idx_hbm.at[wid], idx_v)
        copies = [
            pltpu.async_copy(
                table_hbm.at[idx_v.at[j]],
                rows_v.at[pl.ds(j * CHUNK, CHUNK)],
                sem,
            )
            for j in range(nch)
        ]
        for c in copies:
            c.wait()
        pltpu.sync_copy(rows_v, out_hbm.at[pl.ds(wid * bpw, bpw)])

    return gather_k, nw, nch


TBLK = 2048


NTB = (VOCAB + TBLK - 1) // TBLK


def _tpack_body(src_ref, o_ref):
    x = src_ref[...]
    eye = (
        lax.broadcasted_iota(jnp.int32, (D, D), 0)
        == lax.broadcasted_iota(jnp.int32, (D, D), 1)
    ).astype(jnp.float32)
    dn = (((0,), (0,)), ((), ()))
    lo = lax.dot_general(
        x[:, : TBLK // 2], eye, dn, preferred_element_type=jnp.float32)
    hi = lax.dot_general(
        x[:, TBLK // 2:], eye, dn, preferred_element_type=jnp.float32)
    o_ref[...] = jnp.concatenate([lo, hi], axis=1)


def _pack_rows(embT):
    return pl.pallas_call(
        _tpack_body,
        grid=(NTB,),
        in_specs=[pl.BlockSpec((D, TBLK), lambda i: (0, i))],
        out_specs=pl.BlockSpec((TBLK // 2, FD), lambda i: (i, 0)),
        out_shape=jax.ShapeDtypeStruct((NTB * (TBLK // 2), FD), jnp.float32),
    )(embT)


BLK = 2048


def _proj_body(ff_ref, w_ref, b_ref, g_ref, par_ref, o_ref):
    pairs = g_ref[...]
    odd = (par_ref[...] != 0)
    sel = jnp.where(odd, pairs[:, D:], pairs[:, :D])
    o_ref[...] = (
        jnp.dot(ff_ref[...], w_ref[...], preferred_element_type=jnp.float32)
        + b_ref[...]
        + sel
    )


def kernel(float_feat, id_feat, W, b, emb_table):
    gather_k, nw, nch = _make_gather()
    idx = id_feat.astype(jnp.int32)
    pair_idx = (((idx >> 11) << 10) | (idx & 1023)).reshape(nw, nch, CHUNK)
    parity = ((idx >> 10) & 1).reshape(N, 1)
    packed = _pack_rows(emb_table.T)
    rows = gather_k(packed, pair_idx)
    return pl.pallas_call(
        _proj_body,
        grid=(N // BLK,),
        in_specs=[
            pl.BlockSpec((BLK, FD), lambda i: (i, 0)),
            pl.BlockSpec((FD, D), lambda i: (0, 0)),
            pl.BlockSpec((1, D), lambda i: (0, 0)),
            pl.BlockSpec((BLK, FD), lambda i: (i, 0)),
            pl.BlockSpec((BLK, 1), lambda i: (i, 0)),
        ],
        out_specs=pl.BlockSpec((BLK, D), lambda i: (i, 0)),
        out_shape=jax.ShapeDtypeStruct((N, D), jnp.float32),
    )(float_feat, W, b.reshape(1, D), rows, parity)


# speedup vs baseline: 1.6505x; 1.6505x over previous
import functools

import jax
import jax.numpy as jnp
from jax import lax
from jax.experimental import pallas as pl
from jax.experimental.pallas import tpu as pltpu
from jax.experimental.pallas import tpu_sc as plsc

N = 16384
D = 64
FD = 128
VOCAB = 1000000
CHUNK = 128


@functools.lru_cache(maxsize=1)
def _make_gather():
    info = plsc.get_sparse_core_info()
    nc, ns = info.num_cores, info.num_subcores
    nw = nc * ns
    bpw = N // nw
    nch = bpw // CHUNK
    mesh = plsc.VectorSubcoreMesh(core_axis_name="c", subcore_axis_name="s")

    @functools.partial(
        pl.kernel,
        mesh=mesh,
        out_type=jax.ShapeDtypeStruct((N, FD), jnp.float32),
        compiler_params=pltpu.CompilerParams(use_tc_tiling_on_sc=True),
        scratch_types=[
            pltpu.VMEM((nch, CHUNK), jnp.int32),
            pltpu.VMEM((bpw, FD), jnp.float32),
            pltpu.SemaphoreType.DMA,
        ],
    )
    def gather_k(table_hbm, idx_hbm, out_hbm, idx_v, rows_v, sem):
        wid = lax.axis_index("s") * nc + lax.axis_index("c")
        pltpu.sync_copy(idx_hbm.at[wid], idx_v)
        copies = [
            pltpu.async_copy(
                table_hbm.at[idx_v.at[j]],
                rows_v.at[pl.ds(j * CHUNK, CHUNK)],
                sem,
            )
            for j in range(nch)
        ]
        for c in copies:
            c.wait()
        pltpu.sync_copy(rows_v, out_hbm.at[pl.ds(wid * bpw, bpw)])

    return gather_k, nw, nch


TBLK = 8192


NTB = (VOCAB + TBLK - 1) // TBLK


def _tpack_body(src_ref, o_ref):
    x = src_ref[...]
    eye = (
        lax.broadcasted_iota(jnp.int32, (D, D), 0)
        == lax.broadcasted_iota(jnp.int32, (D, D), 1)
    ).astype(jnp.float32)
    dn = (((0,), (0,)), ((), ()))
    lo = lax.dot_general(
        x[:, : TBLK // 2], eye, dn, preferred_element_type=jnp.float32)
    hi = lax.dot_general(
        x[:, TBLK // 2:], eye, dn, preferred_element_type=jnp.float32)
    o_ref[...] = jnp.concatenate([lo, hi], axis=1)


def _pack_rows(embT):
    return pl.pallas_call(
        _tpack_body,
        grid=(NTB,),
        in_specs=[pl.BlockSpec((D, TBLK), lambda i: (0, i))],
        out_specs=pl.BlockSpec((TBLK // 2, FD), lambda i: (i, 0)),
        out_shape=jax.ShapeDtypeStruct((NTB * (TBLK // 2), FD), jnp.float32),
    )(embT)


BLK = 2048


def _proj_body(ff_ref, w_ref, b_ref, g_ref, par_ref, o_ref):
    pairs = g_ref[...]
    odd = (par_ref[...] != 0)
    sel = jnp.where(odd, pairs[:, D:], pairs[:, :D])
    o_ref[...] = (
        jnp.dot(ff_ref[...], w_ref[...], preferred_element_type=jnp.float32)
        + b_ref[...]
        + sel
    )


def kernel(float_feat, id_feat, W, b, emb_table):
    gather_k, nw, nch = _make_gather()
    idx = id_feat.astype(jnp.int32)
    pair_idx = (((idx >> 13) << 12) | (idx & 4095)).reshape(nw, nch, CHUNK)
    parity = ((idx >> 12) & 1).reshape(N, 1)
    packed = _pack_rows(emb_table.T)
    rows = gather_k(packed, pair_idx)
    return pl.pallas_call(
        _proj_body,
        grid=(N // BLK,),
        in_specs=[
            pl.BlockSpec((BLK, FD), lambda i: (i, 0)),
            pl.BlockSpec((FD, D), lambda i: (0, 0)),
            pl.BlockSpec((1, D), lambda i: (0, 0)),
            pl.BlockSpec((BLK, FD), lambda i: (i, 0)),
            pl.BlockSpec((BLK, 1), lambda i: (i, 0)),
        ],
        out_specs=pl.BlockSpec((BLK, D), lambda i: (i, 0)),
        out_shape=jax.ShapeDtypeStruct((N, D), jnp.float32),
    )(float_feat, W, b.reshape(1, D), rows, parity)


# speedup vs baseline: 1.8499x; 1.1208x over previous
import functools

import jax
import jax.numpy as jnp
from jax import lax
from jax.experimental import pallas as pl
from jax.experimental.pallas import tpu as pltpu
from jax.experimental.pallas import tpu_sc as plsc

N = 16384
D = 64
FD = 128
VOCAB = 1000000
CHUNK = 128


@functools.lru_cache(maxsize=1)
def _make_gather():
    info = plsc.get_sparse_core_info()
    nc, ns = info.num_cores, info.num_subcores
    nw = nc * ns
    bpw = N // nw
    nch = bpw // CHUNK
    mesh = plsc.VectorSubcoreMesh(core_axis_name="c", subcore_axis_name="s")

    @functools.partial(
        pl.kernel,
        mesh=mesh,
        out_type=jax.ShapeDtypeStruct((N, FD), jnp.float32),
        compiler_params=pltpu.CompilerParams(use_tc_tiling_on_sc=True),
        scratch_types=[
            pltpu.VMEM((nch, CHUNK), jnp.int32),
            pltpu.VMEM((bpw, FD), jnp.float32),
            pltpu.SemaphoreType.DMA,
        ],
    )
    def gather_k(table_hbm, idx_hbm, out_hbm, idx_v, rows_v, sem):
        wid = lax.axis_index("s") * nc + lax.axis_index("c")
        pltpu.sync_copy(idx_hbm.at[wid], idx_v)
        copies = [
            pltpu.async_copy(
                table_hbm.at[idx_v.at[j]],
                rows_v.at[pl.ds(j * CHUNK, CHUNK)],
                sem,
            )
            for j in range(nch)
        ]
        for c in copies:
            c.wait()
        pltpu.sync_copy(rows_v, out_hbm.at[pl.ds(wid * bpw, bpw)])

    return gather_k, nw, nch


TBLK = 16384


NTB = (VOCAB + TBLK - 1) // TBLK


def _tpack_body(src_ref, o_ref):
    x = src_ref[...]
    eye = (
        lax.broadcasted_iota(jnp.int32, (D, D), 0)
        == lax.broadcasted_iota(jnp.int32, (D, D), 1)
    ).astype(jnp.float32)
    dn = (((0,), (0,)), ((), ()))
    lo = lax.dot_general(
        x[:, : TBLK // 2], eye, dn, preferred_element_type=jnp.float32)
    hi = lax.dot_general(
        x[:, TBLK // 2:], eye, dn, preferred_element_type=jnp.float32)
    o_ref[...] = jnp.concatenate([lo, hi], axis=1)


def _pack_rows(embT):
    return pl.pallas_call(
        _tpack_body,
        grid=(NTB,),
        in_specs=[pl.BlockSpec((D, TBLK), lambda i: (0, i))],
        out_specs=pl.BlockSpec((TBLK // 2, FD), lambda i: (i, 0)),
        out_shape=jax.ShapeDtypeStruct((NTB * (TBLK // 2), FD), jnp.float32),
    )(embT)


BLK = 2048


def _proj_body(ff_ref, w_ref, b_ref, g_ref, par_ref, o_ref):
    pairs = g_ref[...]
    odd = (par_ref[...] != 0)
    sel = jnp.where(odd, pairs[:, D:], pairs[:, :D])
    o_ref[...] = (
        jnp.dot(ff_ref[...], w_ref[...], preferred_element_type=jnp.float32)
        + b_ref[...]
        + sel
    )


def kernel(float_feat, id_feat, W, b, emb_table):
    gather_k, nw, nch = _make_gather()
    idx = id_feat.astype(jnp.int32)
    pair_idx = (((idx >> 14) << 13) | (idx & 8191)).reshape(nw, nch, CHUNK)
    parity = ((idx >> 13) & 1).reshape(N, 1)
    packed = _pack_rows(emb_table.T)
    rows = gather_k(packed, pair_idx)
    return pl.pallas_call(
        _proj_body,
        grid=(N // BLK,),
        in_specs=[
            pl.BlockSpec((BLK, FD), lambda i: (i, 0)),
            pl.BlockSpec((FD, D), lambda i: (0, 0)),
            pl.BlockSpec((1, D), lambda i: (0, 0)),
            pl.BlockSpec((BLK, FD), lambda i: (i, 0)),
            pl.BlockSpec((BLK, 1), lambda i: (i, 0)),
        ],
        out_specs=pl.BlockSpec((BLK, D), lambda i: (i, 0)),
        out_shape=jax.ShapeDtypeStruct((N, D), jnp.float32),
    )(float_feat, W, b.reshape(1, D), rows, parity)


# speedup vs baseline: 1.9506x; 1.0544x over previous
import functools

import jax
import jax.numpy as jnp
from jax import lax
from jax.experimental import pallas as pl
from jax.experimental.pallas import tpu as pltpu
from jax.experimental.pallas import tpu_sc as plsc

N = 16384
D = 64
FD = 128
VOCAB = 1000000
CHUNK = 128


@functools.lru_cache(maxsize=1)
def _make_gather():
    info = plsc.get_sparse_core_info()
    nc, ns = info.num_cores, info.num_subcores
    nw = nc * ns
    bpw = N // nw
    nch = bpw // CHUNK
    mesh = plsc.VectorSubcoreMesh(core_axis_name="c", subcore_axis_name="s")

    @functools.partial(
        pl.kernel,
        mesh=mesh,
        out_type=jax.ShapeDtypeStruct((N, FD), jnp.float32),
        compiler_params=pltpu.CompilerParams(use_tc_tiling_on_sc=True),
        scratch_types=[
            pltpu.VMEM((nch, CHUNK), jnp.int32),
            pltpu.VMEM((bpw, FD), jnp.float32),
            pltpu.SemaphoreType.DMA,
        ],
    )
    def gather_k(table_hbm, idx_hbm, out_hbm, idx_v, rows_v, sem):
        wid = lax.axis_index("s") * nc + lax.axis_index("c")
        pltpu.sync_copy(idx_hbm.at[wid], idx_v)
        copies = [
            pltpu.async_copy(
                table_hbm.at[idx_v.at[j]],
                rows_v.at[pl.ds(j * CHUNK, CHUNK)],
                sem,
            )
            for j in range(nch)
        ]
        for c in copies:
            c.wait()
        pltpu.sync_copy(rows_v, out_hbm.at[pl.ds(wid * bpw, bpw)])

    return gather_k, nw, nch


TBLK = 32768


NTB = (VOCAB + TBLK - 1) // TBLK


def _tpack_body(src_ref, o_ref):
    x = src_ref[...]
    eye = (
        lax.broadcasted_iota(jnp.int32, (D, D), 0)
        == lax.broadcasted_iota(jnp.int32, (D, D), 1)
    ).astype(jnp.float32)
    dn = (((0,), (0,)), ((), ()))
    lo = lax.dot_general(
        x[:, : TBLK // 2], eye, dn, preferred_element_type=jnp.float32)
    hi = lax.dot_general(
        x[:, TBLK // 2:], eye, dn, preferred_element_type=jnp.float32)
    o_ref[...] = jnp.concatenate([lo, hi], axis=1)


def _pack_rows(embT):
    return pl.pallas_call(
        _tpack_body,
        grid=(NTB,),
        in_specs=[pl.BlockSpec((D, TBLK), lambda i: (0, i))],
        out_specs=pl.BlockSpec((TBLK // 2, FD), lambda i: (i, 0)),
        out_shape=jax.ShapeDtypeStruct((NTB * (TBLK // 2), FD), jnp.float32),
    )(embT)


BLK = 2048


def _proj_body(ff_ref, w_ref, b_ref, g_ref, par_ref, o_ref):
    pairs = g_ref[...]
    odd = (par_ref[...] != 0)
    sel = jnp.where(odd, pairs[:, D:], pairs[:, :D])
    o_ref[...] = (
        jnp.dot(ff_ref[...], w_ref[...], preferred_element_type=jnp.float32)
        + b_ref[...]
        + sel
    )


def kernel(float_feat, id_feat, W, b, emb_table):
    gather_k, nw, nch = _make_gather()
    idx = id_feat.astype(jnp.int32)
    pair_idx = (((idx >> 15) << 14) | (idx & 16383)).reshape(nw, nch, CHUNK)
    parity = ((idx >> 14) & 1).reshape(N, 1)
    packed = _pack_rows(emb_table.T)
    rows = gather_k(packed, pair_idx)
    return pl.pallas_call(
        _proj_body,
        grid=(N // BLK,),
        in_specs=[
            pl.BlockSpec((BLK, FD), lambda i: (i, 0)),
            pl.BlockSpec((FD, D), lambda i: (0, 0)),
            pl.BlockSpec((1, D), lambda i: (0, 0)),
            pl.BlockSpec((BLK, FD), lambda i: (i, 0)),
            pl.BlockSpec((BLK, 1), lambda i: (i, 0)),
        ],
        out_specs=pl.BlockSpec((BLK, D), lambda i: (i, 0)),
        out_shape=jax.ShapeDtypeStruct((N, D), jnp.float32),
    )(float_feat, W, b.reshape(1, D), rows, parity)
